# baseline (speedup 1.0000x reference)
import functools

import jax
import jax.numpy as jnp
from jax import lax
from jax.experimental import pallas as pl
from jax.experimental.pallas import tpu as pltpu
from jax.experimental.pallas import tpu_sc as plsc

N = 10000
E = 320000
D = 128

NC = 2
NS = 16
NW = NC * NS
CHUNK = 128
CHUNKS_PER_TILE = 80
E_PAD = NW * CHUNKS_PER_TILE * CHUNK
TOT_CHUNKS = E_PAD // CHUNK
K0 = TOT_CHUNKS // NS
K1 = 0
SP = 32
N_ACC = 10240
ROWS_PER_TILE = N_ACC // NS
CNT_W = 128

@functools.lru_cache(maxsize=None)
def _sc_kernels():
    mesh = plsc.VectorSubcoreMesh(core_axis_name="c", subcore_axis_name="s",
                                  num_cores=NC, num_subcores=NS)
    mesh1 = plsc.VectorSubcoreMesh(core_axis_name="c", subcore_axis_name="s",
                                   num_cores=1, num_subcores=NS)

    @functools.partial(
        pl.kernel,
        out_type=jax.ShapeDtypeStruct((NC, N_ACC, CNT_W), jnp.float32),
        mesh=mesh,
        scratch_types=[
            pltpu.VMEM((CHUNKS_PER_TILE, CHUNK), jnp.int32),
            pltpu.VMEM((CHUNK, CNT_W), jnp.float32),
            pltpu.VMEM_SHARED((N_ACC, CNT_W), jnp.float32),
        ],
        name="sage_counts_sc",
    )
    def _counts(dstg, zcnt, ones_h, cnt_out, dst_v, ones_v, cnt_sh):
        c = lax.axis_index("c")
        s = lax.axis_index("s")
        base = (c * NS + s) * CHUNKS_PER_TILE
        pltpu.sync_copy(dstg.at[pl.ds(base, CHUNKS_PER_TILE)], dst_v)
        pltpu.sync_copy(ones_h, ones_v)
        row0 = s * ROWS_PER_TILE
        pltpu.sync_copy(zcnt, cnt_sh.at[pl.ds(row0, ROWS_PER_TILE)])
        plsc.subcore_barrier()

        def body(j, carry):
            pltpu.sync_copy(ones_v, cnt_sh.at[dst_v.at[j]], add=True)
            return carry

        lax.fori_loop(0, CHUNKS_PER_TILE, body, 0)
        plsc.subcore_barrier()
        pltpu.sync_copy(cnt_sh.at[pl.ds(row0, ROWS_PER_TILE)],
                        cnt_out.at[c, pl.ds(row0, ROWS_PER_TILE)])

    @functools.partial(
        pl.kernel,
        out_type=jax.ShapeDtypeStruct((NC, N_ACC, D), jnp.float32),
        mesh=mesh,
        scratch_types=[
            pltpu.VMEM((SP, CHUNK), jnp.int32),
            pltpu.VMEM((SP, CHUNK), jnp.int32),
            pltpu.VMEM((2 * CHUNK, D), jnp.float32),
            pltpu.VMEM_SHARED((N_ACC, D), jnp.float32),
            pltpu.SemaphoreType.DMA,
            pltpu.SemaphoreType.DMA,
        ],
        name="sage_agg_sc",
    )
    def _agg(h_hbm, srcg, dstg, sum_out,
             src_v, dst_v, gbuf, acc_sh, sem0, sem1):
        gbuf0 = gbuf.at[pl.ds(0, CHUNK)]
        gbuf1 = gbuf.at[pl.ds(CHUNK, CHUNK)]
        c = lax.axis_index("c")
        s = lax.axis_index("s")

        zero = jnp.zeros((16,), jnp.float32)

        def zrow(i, carry):
            for k in range(D // 16):
                gbuf[i, pl.ds(16 * k, 16)] = zero
            return carry

        lax.fori_loop(0, CHUNK, zrow, 0)
        row0 = s * ROWS_PER_TILE
        for i in range(ROWS_PER_TILE // CHUNK):
            pltpu.sync_copy(gbuf0, acc_sh.at[pl.ds(row0 + CHUNK * i, CHUNK)])
        plsc.subcore_barrier()

        base = s * K0
        npass = jnp.where(c == 0, K0 // SP, 0)

        def ppass(p, carry):
            off = base + p * SP
            pltpu.sync_copy(srcg.at[pl.ds(off, SP)], src_v)
            pltpu.sync_copy(dstg.at[pl.ds(off, SP)], dst_v)
            pltpu.async_copy(h_hbm.at[src_v.at[0]], gbuf0, sem0)

            def body(t, carry2):
                j = 2 * t
                pltpu.async_copy(h_hbm.at[src_v.at[j + 1]], gbuf1, sem1)
                pltpu.make_async_copy(h_hbm.at[src_v.at[j]], gbuf0,
                                      sem0).wait()
                pltpu.sync_copy(gbuf0, acc_sh.at[dst_v.at[j]], add=True)

                @pl.when(t < SP // 2 - 1)
                def _():
                    pltpu.async_copy(h_hbm.at[src_v.at[j + 2]], gbuf0, sem0)

                pltpu.make_async_copy(h_hbm.at[src_v.at[j + 1]], gbuf1,
                                      sem1).wait()
                pltpu.sync_copy(gbuf1, acc_sh.at[dst_v.at[j + 1]], add=True)
                return carry2

            lax.fori_loop(0, SP // 2, body, 0)
            return carry

        lax.fori_loop(0, npass, ppass, 0)
        plsc.subcore_barrier()

        pltpu.sync_copy(acc_sh.at[pl.ds(row0, ROWS_PER_TILE)],
                        sum_out.at[c, pl.ds(row0, ROWS_PER_TILE)])

    return _counts, _agg


RB = 1000
NB = N // RB


def _mm_stats_body(sum_ref, cnt_ref, h_ref, wlt_ref, bl_ref, wrt_ref,
                   y_ref, stats_ref):
    i = pl.program_id(0)
    ssum = sum_ref[0] + sum_ref[1]
    cnt = cnt_ref[0, :, 0:1] + cnt_ref[1, :, 0:1]
    mean = ssum / jnp.maximum(cnt, 1.0)
    y = (jax.lax.dot(mean, wlt_ref[...],
                     precision=jax.lax.Precision.HIGHEST,
                     preferred_element_type=jnp.float32)
         + bl_ref[...][None, :]
         + jax.lax.dot(h_ref[...], wrt_ref[...],
                       precision=jax.lax.Precision.HIGHEST,
                       preferred_element_type=jnp.float32))
    y_ref[...] = y

    @pl.when(i == 0)
    def _():
        stats_ref[...] = jnp.zeros_like(stats_ref)

    stats_ref[0:1, :] += jnp.sum(y, axis=0, keepdims=True)
    stats_ref[1:2, :] += jnp.sum(y * y, axis=0, keepdims=True)


def _bn_body(relu, y_ref, stats_ref, g_ref, b_ref, out_ref):
    mu = stats_ref[0:1, :] * (1.0 / N)
    var = stats_ref[1:2, :] * (1.0 / N) - mu * mu
    y = y_ref[...]
    out = g_ref[...][None, :] * (y - mu) * jax.lax.rsqrt(var + 1e-5) \
        + b_ref[...][None, :]
    if relu:
        out = jnp.maximum(out, 0.0)
    out_ref[...] = out


def _dense(sum_p, cnt_p, h, wlt, bl, wrt, gamma, beta, relu, name):
    y, stats = pl.pallas_call(
        _mm_stats_body,
        grid=(NB,),
        in_specs=[
            pl.BlockSpec((NC, RB, D), lambda i: (0, i, 0)),
            pl.BlockSpec((NC, RB, CNT_W), lambda i: (0, i, 0)),
            pl.BlockSpec((RB, D), lambda i: (i, 0)),
            pl.BlockSpec((D, D), lambda i: (0, 0)),
            pl.BlockSpec((D,), lambda i: (0,)),
            pl.BlockSpec((D, D), lambda i: (0, 0)),
        ],
        out_specs=[
            pl.BlockSpec((RB, D), lambda i: (i, 0)),
            pl.BlockSpec((8, D), lambda i: (0, 0)),
        ],
        out_shape=[jax.ShapeDtypeStruct((N, D), jnp.float32),
                   jax.ShapeDtypeStruct((8, D), jnp.float32)],
        name=name + "_mm",
    )(sum_p, cnt_p, h, wlt, bl, wrt)
    return pl.pallas_call(
        functools.partial(_bn_body, relu),
        grid=(NB,),
        in_specs=[
            pl.BlockSpec((RB, D), lambda i: (i, 0)),
            pl.BlockSpec((8, D), lambda i: (0, 0)),
            pl.BlockSpec((D,), lambda i: (0,)),
            pl.BlockSpec((D,), lambda i: (0,)),
        ],
        out_specs=pl.BlockSpec((RB, D), lambda i: (i, 0)),
        out_shape=jax.ShapeDtypeStruct((N, D), jnp.float32),
        name=name + "_bn",
    )(y, stats, gamma, beta)


def kernel(x, edge_index, Wl0, bl0, Wr0, gamma0, beta0,
           Wl1, bl1, Wr1, gamma1, beta1):
    src = edge_index[0]
    dst = edge_index[1]
    pad = E_PAD - E
    srcg = jnp.concatenate(
        [src, jnp.zeros((pad,), jnp.int32)]).reshape(TOT_CHUNKS, CHUNK)
    dstg = jnp.concatenate(
        [dst, jnp.full((pad,), N, jnp.int32)]).reshape(TOT_CHUNKS, CHUNK)
    zcnt = jnp.zeros((ROWS_PER_TILE, CNT_W), jnp.float32)
    ones = jnp.ones((CHUNK, CNT_W), jnp.float32)

    counts_k, agg_k = _sc_kernels()
    cnt = counts_k(dstg, zcnt, ones)
    sum0 = agg_k(x, srcg, dstg)
    h1 = _dense(sum0, cnt, x, Wl0.T, bl0, Wr0.T, gamma0, beta0, True,
                "sage_dense0_tc")
    sum1 = agg_k(h1, srcg, dstg)
    out = _dense(sum1, cnt, h1, Wl1.T, bl1, Wr1.T, gamma1, beta1, False,
                 "sage_dense1_tc")
    return out


# speedup vs baseline: 2.9586x; 2.9586x over previous
import functools

import jax
import jax.numpy as jnp
from jax import lax
from jax.experimental import pallas as pl
from jax.experimental.pallas import tpu as pltpu
from jax.experimental.pallas import tpu_sc as plsc

N = 10000
E = 320000
D = 128

NC = 2
NS = 16
NW = NC * NS
CHUNK = 128
CHUNKS_PER_TILE = 80
E_PAD = NW * CHUNKS_PER_TILE * CHUNK
TOT_CHUNKS = E_PAD // CHUNK
K = CHUNKS_PER_TILE
SP = 40
N_ACC = 10240
ROWS_PER_TILE = N_ACC // NS
CNT_W = 128

@functools.lru_cache(maxsize=None)
def _sc_kernels():
    mesh = plsc.VectorSubcoreMesh(core_axis_name="c", subcore_axis_name="s",
                                  num_cores=NC, num_subcores=NS)

    @functools.partial(
        pl.kernel,
        out_type=jax.ShapeDtypeStruct((NC, N_ACC, CNT_W), jnp.float32),
        mesh=mesh,
        scratch_types=[
            pltpu.VMEM((CHUNKS_PER_TILE, CHUNK), jnp.int32),
            pltpu.VMEM((CHUNK, CNT_W), jnp.float32),
            pltpu.VMEM_SHARED((N_ACC, CNT_W), jnp.float32),
        ],
        name="sage_counts_sc",
    )
    def _counts(dstg, zcnt, ones_h, cnt_out, dst_v, ones_v, cnt_sh):
        c = lax.axis_index("c")
        s = lax.axis_index("s")
        base = (c * NS + s) * CHUNKS_PER_TILE
        pltpu.sync_copy(dstg.at[pl.ds(base, CHUNKS_PER_TILE)], dst_v)
        pltpu.sync_copy(ones_h, ones_v)
        row0 = s * ROWS_PER_TILE
        pltpu.sync_copy(zcnt, cnt_sh.at[pl.ds(row0, ROWS_PER_TILE)])
        plsc.subcore_barrier()

        def body(j, carry):
            pltpu.sync_copy(ones_v, cnt_sh.at[dst_v.at[j]], add=True)
            return carry

        lax.fori_loop(0, CHUNKS_PER_TILE, body, 0)
        plsc.subcore_barrier()
        pltpu.sync_copy(cnt_sh.at[pl.ds(row0, ROWS_PER_TILE)],
                        cnt_out.at[c, pl.ds(row0, ROWS_PER_TILE)])

    @functools.partial(
        pl.kernel,
        out_type=jax.ShapeDtypeStruct((NC, N_ACC, D), jnp.float32),
        mesh=mesh,
        scratch_types=[
            pltpu.VMEM((SP, CHUNK), jnp.int32),
            pltpu.VMEM((SP, CHUNK), jnp.int32),
            pltpu.VMEM((2 * CHUNK, D), jnp.float32),
            pltpu.VMEM_SHARED((N_ACC, D), jnp.float32),
            pltpu.SemaphoreType.DMA,
            pltpu.SemaphoreType.DMA,
        ],
        name="sage_agg_sc",
    )
    def _agg(h_hbm, srcg, dstg, sum_out,
             src_v, dst_v, gbuf, acc_sh, sem0, sem1):
        gbuf0 = gbuf.at[pl.ds(0, CHUNK)]
        gbuf1 = gbuf.at[pl.ds(CHUNK, CHUNK)]
        c = lax.axis_index("c")
        s = lax.axis_index("s")

        zero = jnp.zeros((16,), jnp.float32)

        def zrow(i, carry):
            for k in range(D // 16):
                gbuf[i, pl.ds(16 * k, 16)] = zero
            return carry

        lax.fori_loop(0, CHUNK, zrow, 0)
        row0 = s * ROWS_PER_TILE
        for i in range(ROWS_PER_TILE // CHUNK):
            pltpu.sync_copy(gbuf0, acc_sh.at[pl.ds(row0 + CHUNK * i, CHUNK)])
        plsc.subcore_barrier()

        base = (c * NS + s) * K
        npass = K // SP

        def ppass(p, carry):
            off = base + p * SP
            pltpu.sync_copy(srcg.at[pl.ds(off, SP)], src_v)
            pltpu.sync_copy(dstg.at[pl.ds(off, SP)], dst_v)
            pltpu.async_copy(h_hbm.at[src_v.at[0]], gbuf0, sem0)

            def body(t, carry2):
                j = 2 * t
                pltpu.async_copy(h_hbm.at[src_v.at[j + 1]], gbuf1, sem1)
                pltpu.make_async_copy(h_hbm.at[src_v.at[j]], gbuf0,
                                      sem0).wait()
                pltpu.sync_copy(gbuf0, acc_sh.at[dst_v.at[j]], add=True)

                @pl.when(t < SP // 2 - 1)
                def _():
                    pltpu.async_copy(h_hbm.at[src_v.at[j + 2]], gbuf0, sem0)

                pltpu.make_async_copy(h_hbm.at[src_v.at[j + 1]], gbuf1,
                                      sem1).wait()
                pltpu.sync_copy(gbuf1, acc_sh.at[dst_v.at[j + 1]], add=True)
                return carry2

            lax.fori_loop(0, SP // 2, body, 0)
            return carry

        lax.fori_loop(0, npass, ppass, 0)
        plsc.subcore_barrier()

        pltpu.sync_copy(acc_sh.at[pl.ds(row0, ROWS_PER_TILE)],
                        sum_out.at[c, pl.ds(row0, ROWS_PER_TILE)])

    return _counts, _agg


RB = 1000
NB = N // RB


def _mm_stats_body(sum_ref, cnt_ref, h_ref, wlt_ref, bl_ref, wrt_ref,
                   y_ref, stats_ref):
    i = pl.program_id(0)
    ssum = sum_ref[0] + sum_ref[1]
    cnt = cnt_ref[0, :, 0:1] + cnt_ref[1, :, 0:1]
    mean = ssum / jnp.maximum(cnt, 1.0)
    y = (jax.lax.dot(mean, wlt_ref[...],
                     precision=jax.lax.Precision.HIGHEST,
                     preferred_element_type=jnp.float32)
         + bl_ref[...][None, :]
         + jax.lax.dot(h_ref[...], wrt_ref[...],
                       precision=jax.lax.Precision.HIGHEST,
                       preferred_element_type=jnp.float32))
    y_ref[...] = y

    @pl.when(i == 0)
    def _():
        stats_ref[...] = jnp.zeros_like(stats_ref)

    stats_ref[0:1, :] += jnp.sum(y, axis=0, keepdims=True)
    stats_ref[1:2, :] += jnp.sum(y * y, axis=0, keepdims=True)


def _bn_body(relu, y_ref, stats_ref, g_ref, b_ref, out_ref):
    mu = stats_ref[0:1, :] * (1.0 / N)
    var = stats_ref[1:2, :] * (1.0 / N) - mu * mu
    y = y_ref[...]
    out = g_ref[...][None, :] * (y - mu) * jax.lax.rsqrt(var + 1e-5) \
        + b_ref[...][None, :]
    if relu:
        out = jnp.maximum(out, 0.0)
    out_ref[...] = out


def _dense(sum_p, cnt_p, h, wlt, bl, wrt, gamma, beta, relu, name):
    y, stats = pl.pallas_call(
        _mm_stats_body,
        grid=(NB,),
        in_specs=[
            pl.BlockSpec((NC, RB, D), lambda i: (0, i, 0)),
            pl.BlockSpec((NC, RB, CNT_W), lambda i: (0, i, 0)),
            pl.BlockSpec((RB, D), lambda i: (i, 0)),
            pl.BlockSpec((D, D), lambda i: (0, 0)),
            pl.BlockSpec((D,), lambda i: (0,)),
            pl.BlockSpec((D, D), lambda i: (0, 0)),
        ],
        out_specs=[
            pl.BlockSpec((RB, D), lambda i: (i, 0)),
            pl.BlockSpec((8, D), lambda i: (0, 0)),
        ],
        out_shape=[jax.ShapeDtypeStruct((N, D), jnp.float32),
                   jax.ShapeDtypeStruct((8, D), jnp.float32)],
        name=name + "_mm",
    )(sum_p, cnt_p, h, wlt, bl, wrt)
    return pl.pallas_call(
        functools.partial(_bn_body, relu),
        grid=(NB,),
        in_specs=[
            pl.BlockSpec((RB, D), lambda i: (i, 0)),
            pl.BlockSpec((8, D), lambda i: (0, 0)),
            pl.BlockSpec((D,), lambda i: (0,)),
            pl.BlockSpec((D,), lambda i: (0,)),
        ],
        out_specs=pl.BlockSpec((RB, D), lambda i: (i, 0)),
        out_shape=jax.ShapeDtypeStruct((N, D), jnp.float32),
        name=name + "_bn",
    )(y, stats, gamma, beta)


def kernel(x, edge_index, Wl0, bl0, Wr0, gamma0, beta0,
           Wl1, bl1, Wr1, gamma1, beta1):
    src = edge_index[0]
    dst = edge_index[1]
    pad = E_PAD - E
    ar = jnp.arange(pad, dtype=jnp.int32)
    srcg = jnp.concatenate(
        [src, (ar * 37) % N]).reshape(TOT_CHUNKS, CHUNK)
    dstg = jnp.concatenate(
        [dst, N + (ar % (N_ACC - N))]).reshape(TOT_CHUNKS, CHUNK)
    zcnt = jnp.zeros((ROWS_PER_TILE, CNT_W), jnp.float32)
    ones = jnp.ones((CHUNK, CNT_W), jnp.float32)

    counts_k, agg_k = _sc_kernels()
    cnt = counts_k(dstg, zcnt, ones)
    sum0 = agg_k(x, srcg, dstg)
    h1 = _dense(sum0, cnt, x, Wl0.T, bl0, Wr0.T, gamma0, beta0, True,
                "sage_dense0_tc")
    sum1 = agg_k(h1, srcg, dstg)
    out = _dense(sum1, cnt, h1, Wl1.T, bl1, Wr1.T, gamma1, beta1, False,
                 "sage_dense1_tc")
    return out
